# speedup vs baseline: 1.2601x; 1.2601x over previous
import functools

import jax
import jax.numpy as jnp
from jax import lax
from jax.experimental import pallas as pl
from jax.experimental.pallas import tpu as pltpu
from jax.experimental.pallas import tpu_sc as plsc

_D = 768
_NW = 32
_C = 64
_N_CHUNKS = 40


def _emb_body(idx_hbm, table_hbm, out_hbm, idx_v, rows_v, sem):
    wid = lax.axis_index("s") * 2 + lax.axis_index("c")
    pltpu.sync_copy(idx_hbm.at[wid], idx_v)

    @pl.loop(0, _N_CHUNKS)
    def _(j):
        pltpu.async_copy(table_hbm.at[idx_v.at[j]], rows_v, sem).wait()
        pltpu.sync_copy(rows_v, out_hbm.at[wid].at[j])


@jax.jit
def _emb(idx, weight):
    mesh = plsc.VectorSubcoreMesh(core_axis_name="c", subcore_axis_name="s")
    return pl.kernel(
        _emb_body,
        out_type=jax.ShapeDtypeStruct((_NW, _N_CHUNKS, _C, _D), jnp.float32),
        mesh=mesh,
        scratch_types=[
            pltpu.VMEM((_N_CHUNKS, _C), jnp.int32),
            pltpu.VMEM((_C, _D), jnp.float32),
            pltpu.SemaphoreType.DMA,
        ],
    )(idx, weight)


def kernel(text, weight):
    b, h = text.shape
    idx = text.reshape(_NW, _N_CHUNKS, _C).astype(jnp.int32)
    out = _emb(idx, weight)
    return out.reshape(b, h, _D)


# speedup vs baseline: 1.3022x; 1.0334x over previous
import functools

import jax
import jax.numpy as jnp
from jax import lax
from jax.experimental import pallas as pl
from jax.experimental.pallas import tpu as pltpu
from jax.experimental.pallas import tpu_sc as plsc

_D = 768
_NW = 32
_C = 32
_N_CHUNKS = 80
_NBUF = 4
_LOOK = 2


def _emb_body(idx_hbm, table_hbm, out_hbm, idx_v, *bufs_and_sems):
    bufs = bufs_and_sems[:_NBUF]
    gsems = bufs_and_sems[_NBUF:2 * _NBUF]
    ssems = bufs_and_sems[2 * _NBUF:3 * _NBUF]
    wid = lax.axis_index("s") * 2 + lax.axis_index("c")
    pltpu.sync_copy(idx_hbm.at[wid], idx_v)

    def start_gather(c, b):
        pltpu.async_copy(table_hbm.at[idx_v.at[c]], bufs[b], gsems[b])

    def wait_gather(c, b):
        pltpu.make_async_copy(table_hbm.at[idx_v.at[c]], bufs[b], gsems[b]).wait()

    def start_store(c, b):
        pltpu.async_copy(bufs[b], out_hbm.at[wid].at[c], ssems[b])

    def wait_store(c, b):
        pltpu.make_async_copy(bufs[b], out_hbm.at[wid].at[c], ssems[b]).wait()

    for b in range(_LOOK):
        start_gather(b, b)

    @pl.loop(0, _N_CHUNKS, step=_NBUF)
    def _(j):
        for b in range(_NBUF):
            c = j + b
            bn = (b + _LOOK) % _NBUF

            @pl.when(c + _LOOK < _N_CHUNKS)
            def _():
                @pl.when(c >= _NBUF - _LOOK)
                def _():
                    wait_store(c + _LOOK - _NBUF, bn)
                start_gather(c + _LOOK, bn)

            wait_gather(c, b)
            start_store(c, b)

    for k in range(_NBUF):
        c = _N_CHUNKS - _NBUF + k
        wait_store(c, c % _NBUF)


@jax.jit
def _emb(idx, weight):
    mesh = plsc.VectorSubcoreMesh(core_axis_name="c", subcore_axis_name="s")
    return pl.kernel(
        _emb_body,
        out_type=jax.ShapeDtypeStruct((_NW, _N_CHUNKS, _C, _D), jnp.float32),
        mesh=mesh,
        scratch_types=(
            [pltpu.VMEM((_N_CHUNKS, _C), jnp.int32)]
            + [pltpu.VMEM((_C, _D), jnp.float32) for _ in range(_NBUF)]
            + [pltpu.SemaphoreType.DMA for _ in range(2 * _NBUF)]
        ),
    )(idx, weight)


def kernel(text, weight):
    b, h = text.shape
    idx = text.reshape(_NW, _N_CHUNKS, _C).astype(jnp.int32)
    out = _emb(idx, weight)
    return out.reshape(b, h, _D)


# speedup vs baseline: 4.2470x; 3.2615x over previous
import functools

import jax
import jax.numpy as jnp
from jax import lax
from jax.experimental import pallas as pl
from jax.experimental.pallas import tpu as pltpu
from jax.experimental.pallas import tpu_sc as plsc

_D = 768
_NW = 32
_C = 32
_N_CHUNKS = 80
_NBUF = 4
_LOOK = 2


def _emb_body(idx_hbm, table_hbm, out_hbm, idx_v, *bufs_and_sems):
    bufs = bufs_and_sems[:_NBUF]
    gsems = bufs_and_sems[_NBUF:2 * _NBUF]
    ssems = bufs_and_sems[2 * _NBUF:3 * _NBUF]
    wid = lax.axis_index("s") * 2 + lax.axis_index("c")
    pltpu.sync_copy(idx_hbm.at[wid], idx_v)

    def start_gather(c, b):
        pltpu.async_copy(table_hbm.at[idx_v.at[c]], bufs[b], gsems[b])

    def wait_gather(c, b):
        pltpu.make_async_copy(table_hbm.at[idx_v.at[c]], bufs[b], gsems[b]).wait()

    def start_store(c, b):
        pltpu.async_copy(bufs[b], out_hbm.at[wid].at[c], ssems[b])

    def wait_store(c, b):
        pltpu.make_async_copy(bufs[b], out_hbm.at[wid].at[c], ssems[b]).wait()

    for b in range(_LOOK):
        start_gather(b, b)

    @pl.loop(0, _N_CHUNKS, step=_NBUF)
    def _(j):
        for b in range(_NBUF):
            c = j + b
            bn = (b + _LOOK) % _NBUF

            @pl.when(c + _LOOK < _N_CHUNKS)
            def _():
                @pl.when(c >= _NBUF - _LOOK)
                def _():
                    wait_store(c + _LOOK - _NBUF, bn)
                start_gather(c + _LOOK, bn)

            wait_gather(c, b)
            start_store(c, b)

    for k in range(_NBUF):
        c = _N_CHUNKS - _NBUF + k
        wait_store(c, c % _NBUF)


@jax.jit
def _emb(idx, weight):
    mesh = plsc.VectorSubcoreMesh(core_axis_name="c", subcore_axis_name="s")
    return pl.kernel(
        _emb_body,
        out_type=jax.ShapeDtypeStruct((_NW, _N_CHUNKS, _C, _D), jnp.float32),
        mesh=mesh,
        scratch_types=(
            [pltpu.VMEM((_N_CHUNKS, _C), jnp.int32)]
            + [pltpu.VMEM((_C, _D), jnp.float32) for _ in range(_NBUF)]
            + [pltpu.SemaphoreType.DMA for _ in range(2 * _NBUF)]
        ),
    )(idx, weight)


def kernel(text, weight):
    b, h = text.shape
    idx = text.T.reshape(_NW, _N_CHUNKS, _C).astype(jnp.int32)
    out = _emb(idx, weight)
    return out.reshape(h, b, _D).transpose(1, 0, 2)
